# baseline (speedup 1.0000x reference)
import jax
import jax.numpy as jnp
from jax.experimental import pallas as pl
from jax.experimental.pallas import tpu as pltpu

_B, _N, _K = 64, 196, 4096


def _dense_kernel(mask_ref, ps_ref, pt_ref, num_ref, den_ref):
    ps = ps_ref[0]
    pt = pt_ref[0]
    m = mask_ref[0]
    safe = jnp.where(m > 0.0, ps, jnp.ones_like(ps))
    num_ref[...] = (-jnp.sum(pt * jnp.log(safe) * m)).reshape(1, 1, 1)
    den_ref[...] = jnp.sum(m).reshape(1, 1, 1)


def kernel(ps, pt, bool_masked_pos):
    maskf = bool_masked_pos.astype(jnp.float32)[..., None]
    num, den = pl.pallas_call(
        _dense_kernel,
        grid=(_B,),
        in_specs=[
            pl.BlockSpec((1, _N, 1), lambda i: (i, 0, 0)),
            pl.BlockSpec((1, _N, _K), lambda i: (i, 0, 0)),
            pl.BlockSpec((1, _N, _K), lambda i: (i, 0, 0)),
        ],
        out_specs=[
            pl.BlockSpec((1, 1, 1), lambda i: (i, 0, 0)),
            pl.BlockSpec((1, 1, 1), lambda i: (i, 0, 0)),
        ],
        out_shape=[
            jax.ShapeDtypeStruct((_B, 1, 1), jnp.float32),
            jax.ShapeDtypeStruct((_B, 1, 1), jnp.float32),
        ],
        compiler_params=pltpu.CompilerParams(
            dimension_semantics=("parallel",),
        ),
    )(maskf, ps, pt)
    return jnp.sum(num) / jnp.sum(den)


# speedup vs baseline: 1.0649x; 1.0649x over previous
import jax
import jax.numpy as jnp
from jax.experimental import pallas as pl
from jax.experimental.pallas import tpu as pltpu

_B, _N, _K = 64, 196, 4096


_BB = 4
_GRID = _B // _BB


def _dense_kernel(mask_ref, ps_ref, pt_ref, num_ref, den_ref):
    part = jnp.float32(0.0)
    cnt = jnp.float32(0.0)
    for b in range(_BB):
        ps = ps_ref[b]
        pt = pt_ref[b]
        m = mask_ref[b]
        safe = jnp.where(m > 0.0, ps, jnp.ones_like(ps))
        part += jnp.sum(pt * jnp.log(safe) * m)
        cnt += jnp.sum(m)
    num_ref[...] = (-part).reshape(1, 1, 1)
    den_ref[...] = cnt.reshape(1, 1, 1)


def kernel(ps, pt, bool_masked_pos):
    maskf = bool_masked_pos.astype(jnp.float32)[..., None]
    num, den = pl.pallas_call(
        _dense_kernel,
        grid=(_GRID,),
        in_specs=[
            pl.BlockSpec((_BB, _N, 1), lambda i: (i, 0, 0)),
            pl.BlockSpec((_BB, _N, _K), lambda i: (i, 0, 0)),
            pl.BlockSpec((_BB, _N, _K), lambda i: (i, 0, 0)),
        ],
        out_specs=[
            pl.BlockSpec((1, 1, 1), lambda i: (i, 0, 0)),
            pl.BlockSpec((1, 1, 1), lambda i: (i, 0, 0)),
        ],
        out_shape=[
            jax.ShapeDtypeStruct((_GRID, 1, 1), jnp.float32),
            jax.ShapeDtypeStruct((_GRID, 1, 1), jnp.float32),
        ],
        compiler_params=pltpu.CompilerParams(
            dimension_semantics=("parallel",),
        ),
    )(maskf, ps, pt)
    return jnp.sum(num) / jnp.sum(den)
